# baseline (speedup 1.0000x reference)
import functools

import jax
import jax.numpy as jnp
from jax import lax
from jax.experimental import pallas as pl
from jax.experimental.pallas import tpu as pltpu
from jax.experimental.pallas import tpu_sc as plsc

V_DIM = 1_000_000
EMB = 64
BATCH = 16384
CTX = 20
LANES = 16
ROW_W = 128

NC = 2
NS = 16
NW = NC * NS

BLK_C = 512
NBLK = V_DIM // BLK_C
V_MAIN = NBLK * BLK_C
V_TAIL = V_DIM - V_MAIN
K1_ITERS = -(-NBLK // NW)

B_PER_W = BATCH // NW
T = 32
NCHUNK = B_PER_W // T
IDX_W = 128
IDX_ROWS = B_PER_W * CTX // IDX_W
ROWS_PER_CHUNK = T * CTX
DMA_PER_CHUNK = ROWS_PER_CHUNK // IDX_W


def _transpose_body(tblT_hbm, tail_hbm, wide_hbm, in_v, out_v, sem):
    wid = lax.axis_index("s") * NC + lax.axis_index("c")
    rvecs = [k * LANES + lax.iota(jnp.int32, LANES) for k in range(EMB // LANES)]

    def step(i, carry):
        j = wid + i * NW
        jprev = j - NW

        @pl.when(jnp.logical_and(i > 0, jprev < NBLK))
        def _():
            pltpu.make_async_copy(
                out_v, wide_hbm.at[pl.ds(jprev * BLK_C, BLK_C), :], sem
            ).wait()

        @pl.when(j < NBLK)
        def _():
            pltpu.sync_copy(tblT_hbm.at[:, pl.ds(j * BLK_C, BLK_C)], in_v)

            def row_body(rr, rcarry):
                cvec = lax.broadcast(rr, (LANES,))
                for k in range(EMB // LANES):
                    out_v[rr, pl.ds(k * LANES, LANES)] = plsc.load_gather(
                        in_v, [rvecs[k], cvec]
                    )
                return rcarry

            lax.fori_loop(0, BLK_C, row_body, 0)
            pltpu.async_copy(
                out_v, wide_hbm.at[pl.ds(j * BLK_C, BLK_C), :], sem
            )
        return carry

    lax.fori_loop(0, K1_ITERS, step, 0)

    jlast = wid + (K1_ITERS - 1) * NW

    @pl.when(jlast < NBLK)
    def _():
        pltpu.make_async_copy(
            out_v, wide_hbm.at[pl.ds(jlast * BLK_C, BLK_C), :], sem
        ).wait()

    @pl.when(wid == NW - 1)
    def _():
        pltpu.sync_copy(tail_hbm, wide_hbm.at[pl.ds(V_MAIN, V_TAIL), :])


def _cbow_body(x_hbm, table_hbm, out_hbm, idx_v, rows_v, outc_v, sem):
    wid = lax.axis_index("s") * NC + lax.axis_index("c")
    pltpu.sync_copy(x_hbm.at[pl.ds(wid * IDX_ROWS, IDX_ROWS), :], idx_v)
    out_base = wid * B_PER_W

    def chunk_body(t, carry):
        cps = []
        for j in range(DMA_PER_CHUNK):
            cps.append(
                pltpu.async_copy(
                    table_hbm.at[idx_v.at[t * DMA_PER_CHUNK + j]],
                    rows_v.at[pl.ds(j * IDX_W, IDX_W), :],
                    sem,
                )
            )
        for cp in cps:
            cp.wait()

        def b_body(b, bcarry):
            r0 = b * CTX
            for k in range(EMB // LANES):
                sl = pl.ds(k * LANES, LANES)
                a0 = rows_v[r0 + 0, sl] + rows_v[r0 + 1, sl]
                a1 = rows_v[r0 + 2, sl] + rows_v[r0 + 3, sl]
                a2 = rows_v[r0 + 4, sl] + rows_v[r0 + 5, sl]
                a3 = rows_v[r0 + 6, sl] + rows_v[r0 + 7, sl]
                for c in range(8, CTX, 4):
                    a0 = a0 + rows_v[r0 + c + 0, sl]
                    a1 = a1 + rows_v[r0 + c + 1, sl]
                    a2 = a2 + rows_v[r0 + c + 2, sl]
                    a3 = a3 + rows_v[r0 + c + 3, sl]
                outc_v[b, sl] = ((a0 + a1) + (a2 + a3)) * (1.0 / CTX)
            return bcarry

        lax.fori_loop(0, T, b_body, 0)
        pltpu.sync_copy(outc_v, out_hbm.at[pl.ds(out_base + t * T, T), :])
        return carry

    lax.fori_loop(0, NCHUNK, chunk_body, 0)


def kernel(x, table):
    mesh = plsc.VectorSubcoreMesh(core_axis_name="c", subcore_axis_name="s")

    tail_p = jnp.pad(table[V_MAIN:, :], ((0, 0), (0, ROW_W - EMB)))
    widen = functools.partial(
        pl.kernel,
        mesh=mesh,
        out_type=jax.ShapeDtypeStruct((V_DIM, ROW_W), jnp.float32),
        scratch_types=[
            pltpu.VMEM((EMB, BLK_C), jnp.float32),
            pltpu.VMEM((BLK_C, ROW_W), jnp.float32),
            pltpu.SemaphoreType.DMA,
        ],
        compiler_params=pltpu.CompilerParams(
            use_tc_tiling_on_sc=True, needs_layout_passes=False
        ),
    )(_transpose_body)
    wide = widen(table.T, tail_p)

    x2 = x.reshape(BATCH * CTX // IDX_W, IDX_W)
    run = functools.partial(
        pl.kernel,
        mesh=mesh,
        out_type=jax.ShapeDtypeStruct((BATCH, EMB), jnp.float32),
        scratch_types=[
            pltpu.VMEM((IDX_ROWS, IDX_W), jnp.int32),
            pltpu.VMEM((ROWS_PER_CHUNK, ROW_W), jnp.float32),
            pltpu.VMEM((T, EMB), jnp.float32),
            pltpu.SemaphoreType.DMA,
        ],
    )(_cbow_body)
    return run(x2, wide)


# speedup vs baseline: 3.2295x; 3.2295x over previous
import functools

import jax
import jax.numpy as jnp
from jax import lax
from jax.experimental import pallas as pl
from jax.experimental.pallas import tpu as pltpu
from jax.experimental.pallas import tpu_sc as plsc

V_DIM = 1_000_000
EMB = 64
BATCH = 16384
CTX = 20
LANES = 16
ROW_W = 128

NC = 2
NS = 16
NW = NC * NS

B_PER_W = BATCH // NW
T = 128
NCHUNK = B_PER_W // T


def _cbow_body(xt_hbm, tbl_hbm, out_hbm, xt_v, acc_v, sem):
    wid = lax.axis_index("s") * NC + lax.axis_index("c")
    base = wid * B_PER_W
    pltpu.sync_copy(xt_hbm.at[:, pl.ds(base, B_PER_W)], xt_v)

    zero = lax.broadcast(jnp.float32(0.0), (LANES,))

    def chunk_body(t, carry):
        def z_body(rr, zcarry):
            for k in range(EMB // LANES):
                acc_v[rr, pl.ds(k * LANES, LANES)] = zero
            return zcarry

        lax.fori_loop(0, T, z_body, 0)

        cps = [
            pltpu.async_copy(
                tbl_hbm.at[xt_v.at[c, pl.ds(t * T, T)]],
                acc_v,
                sem,
                add=True,
            )
            for c in range(CTX)
        ]
        for cp in cps:
            cp.wait()

        def s_body(rr, scarry):
            for k in range(EMB // LANES):
                sl = pl.ds(k * LANES, LANES)
                acc_v[rr, sl] = acc_v[rr, sl] * (1.0 / CTX)
            return scarry

        lax.fori_loop(0, T, s_body, 0)
        pltpu.sync_copy(acc_v, out_hbm.at[pl.ds(base + t * T, T), :])
        return carry

    lax.fori_loop(0, NCHUNK, chunk_body, 0)


def kernel(x, table):
    tblp = jnp.pad(table, ((0, 0), (0, ROW_W - EMB)))
    tbl2 = tblp.reshape(2 * V_DIM, EMB)
    xt = (x * 2).T

    mesh = plsc.VectorSubcoreMesh(core_axis_name="c", subcore_axis_name="s")
    run = functools.partial(
        pl.kernel,
        mesh=mesh,
        out_type=jax.ShapeDtypeStruct((BATCH, EMB), jnp.float32),
        scratch_types=[
            pltpu.VMEM((CTX, B_PER_W), jnp.int32),
            pltpu.VMEM((T, EMB), jnp.float32),
            pltpu.SemaphoreType.DMA,
        ],
        compiler_params=pltpu.CompilerParams(use_tc_tiling_on_sc=False),
    )(_cbow_body)
    return run(xt, tbl2)
